# baseline (speedup 1.0000x reference)
import functools

import jax
import jax.numpy as jnp
from jax import lax
from jax.experimental import pallas as pl
from jax.experimental.pallas import tpu as pltpu

_BLK_K = 2048
_NBUF = 3
_NSPLIT = 2


def _chunk_copies(at_hbm, abuf, sems, j, slot):
    rows = _BLK_K // _NSPLIT
    return [
        pltpu.make_async_copy(
            at_hbm.at[pl.ds(j * _BLK_K + s * rows, rows), :],
            abuf.at[slot, pl.ds(s * rows, rows), :],
            sems.at[slot, s],
        )
        for s in range(_NSPLIT)
    ]


def _mm_kernel(bt_ref, at_hbm, o_ref, abuf, tbuf, sems, tsem, *, nchunks,
               tail):
    n, m = o_ref.shape

    tail_cp = pltpu.make_async_copy(
        at_hbm.at[pl.ds(nchunks * _BLK_K, tail), :],
        tbuf.at[pl.ds(0, tail), :],
        tsem,
    )
    tail_cp.start()
    for j in range(_NBUF - 1):
        for c in _chunk_copies(at_hbm, abuf, sems, j, j):
            c.start()

    def body(j, acc):
        nxt = j + _NBUF - 1

        @pl.when(nxt < nchunks)
        def _prefetch():
            for c in _chunk_copies(at_hbm, abuf, sems, nxt,
                                   lax.rem(nxt, _NBUF)):
                c.start()

        slot = lax.rem(j, _NBUF)
        for c in _chunk_copies(at_hbm, abuf, sems, j, slot):
            c.wait()
        bt = bt_ref[:, pl.ds(j * _BLK_K, _BLK_K)]
        return acc + jnp.dot(
            bt.astype(jnp.bfloat16),
            abuf[slot].astype(jnp.bfloat16),
            preferred_element_type=jnp.float32,
        )

    acc = lax.fori_loop(0, nchunks, body, jnp.zeros((n, m), jnp.float32))

    tail_cp.wait()
    bt_t = bt_ref[:, pl.ds(nchunks * _BLK_K, tail)]
    acc += jnp.dot(
        bt_t.astype(jnp.bfloat16),
        tbuf[pl.ds(0, tail), :].astype(jnp.bfloat16),
        preferred_element_type=jnp.float32,
    )
    o_ref[...] = acc


def kernel(attention_weights, content_matrix):
    m, k_dim = attention_weights.shape
    _, n = content_matrix.shape
    at = attention_weights.T
    bt = content_matrix.T
    nchunks = k_dim // _BLK_K
    tail = k_dim - nchunks * _BLK_K
    body = functools.partial(_mm_kernel, nchunks=nchunks, tail=tail)
    out_t = pl.pallas_call(
        body,
        grid=(1,),
        in_specs=[
            pl.BlockSpec((n, k_dim), lambda i: (0, 0)),
            pl.BlockSpec(memory_space=pltpu.MemorySpace.HBM),
        ],
        out_specs=pl.BlockSpec((n, m), lambda i: (0, 0)),
        out_shape=jax.ShapeDtypeStruct((n, m), jnp.float32),
        scratch_shapes=[
            pltpu.VMEM((_NBUF, _BLK_K, m), jnp.float32),
            pltpu.VMEM((tail, m), jnp.float32),
            pltpu.SemaphoreType.DMA((_NBUF, _NSPLIT)),
            pltpu.SemaphoreType.DMA,
        ],
    )(bt, at)
    return out_t.T


# speedup vs baseline: 1.0040x; 1.0040x over previous
import functools

import jax
import jax.numpy as jnp
from jax import lax
from jax.experimental import pallas as pl
from jax.experimental.pallas import tpu as pltpu

_BLK_K = 1024
_NBUF = 6
_NSPLIT = 1


def _chunk_copies(at_hbm, abuf, sems, j, slot):
    rows = _BLK_K // _NSPLIT
    return [
        pltpu.make_async_copy(
            at_hbm.at[pl.ds(j * _BLK_K + s * rows, rows), :],
            abuf.at[slot, pl.ds(s * rows, rows), :],
            sems.at[slot, s],
        )
        for s in range(_NSPLIT)
    ]


def _mm_kernel(bt_ref, at_hbm, o_ref, abuf, tbuf, sems, tsem, *, nchunks,
               tail):
    n, m = o_ref.shape

    tail_cp = pltpu.make_async_copy(
        at_hbm.at[pl.ds(nchunks * _BLK_K, tail), :],
        tbuf.at[pl.ds(0, tail), :],
        tsem,
    )
    tail_cp.start()
    for j in range(_NBUF - 1):
        for c in _chunk_copies(at_hbm, abuf, sems, j, j):
            c.start()

    def body(j, acc):
        nxt = j + _NBUF - 1

        @pl.when(nxt < nchunks)
        def _prefetch():
            for c in _chunk_copies(at_hbm, abuf, sems, nxt,
                                   lax.rem(nxt, _NBUF)):
                c.start()

        slot = lax.rem(j, _NBUF)
        for c in _chunk_copies(at_hbm, abuf, sems, j, slot):
            c.wait()
        bt = bt_ref[:, pl.ds(j * _BLK_K, _BLK_K)]
        return acc + jnp.dot(
            bt.astype(jnp.bfloat16),
            abuf[slot].astype(jnp.bfloat16),
            preferred_element_type=jnp.float32,
        )

    acc = lax.fori_loop(0, nchunks, body, jnp.zeros((n, m), jnp.float32))

    tail_cp.wait()
    bt_t = bt_ref[:, pl.ds(nchunks * _BLK_K, tail)]
    acc += jnp.dot(
        bt_t.astype(jnp.bfloat16),
        tbuf[pl.ds(0, tail), :].astype(jnp.bfloat16),
        preferred_element_type=jnp.float32,
    )
    o_ref[...] = acc


def kernel(attention_weights, content_matrix):
    m, k_dim = attention_weights.shape
    _, n = content_matrix.shape
    at = attention_weights.T
    bt = content_matrix.T
    nchunks = k_dim // _BLK_K
    tail = k_dim - nchunks * _BLK_K
    body = functools.partial(_mm_kernel, nchunks=nchunks, tail=tail)
    out_t = pl.pallas_call(
        body,
        grid=(1,),
        in_specs=[
            pl.BlockSpec((n, k_dim), lambda i: (0, 0)),
            pl.BlockSpec(memory_space=pltpu.MemorySpace.HBM),
        ],
        out_specs=pl.BlockSpec((n, m), lambda i: (0, 0)),
        out_shape=jax.ShapeDtypeStruct((n, m), jnp.float32),
        scratch_shapes=[
            pltpu.VMEM((_NBUF, _BLK_K, m), jnp.float32),
            pltpu.VMEM((tail, m), jnp.float32),
            pltpu.SemaphoreType.DMA((_NBUF, _NSPLIT)),
            pltpu.SemaphoreType.DMA,
        ],
    )(bt, at)
    return out_t.T


# speedup vs baseline: 1.0107x; 1.0066x over previous
import functools

import jax
import jax.numpy as jnp
from jax import lax
from jax.experimental import pallas as pl
from jax.experimental.pallas import tpu as pltpu

_BLK_K = 2560


def _mm_kernel(bt_ref, at_ref, o_ref, *, nsteps, tail):
    k = pl.program_id(0)

    @pl.when(k == 0)
    def _init():
        o_ref[...] = jnp.zeros_like(o_ref)

    @pl.when(k < nsteps - 1)
    def _body():
        o_ref[...] += jnp.dot(
            bt_ref[...].astype(jnp.bfloat16),
            at_ref[...].astype(jnp.bfloat16),
            preferred_element_type=jnp.float32,
        )

    @pl.when(k == nsteps - 1)
    def _tail():
        bt = bt_ref[...]
        col = lax.broadcasted_iota(jnp.int32, bt.shape, 1)
        bt = jnp.where(col < tail, bt, 0.0)
        at = at_ref[...]
        row = lax.broadcasted_iota(jnp.int32, at.shape, 0)
        at = jnp.where(row < tail, at, 0.0)
        o_ref[...] += jnp.dot(
            bt.astype(jnp.bfloat16),
            at.astype(jnp.bfloat16),
            preferred_element_type=jnp.float32,
        )


def kernel(attention_weights, content_matrix):
    m, k_dim = attention_weights.shape
    _, n = content_matrix.shape
    at = attention_weights.T
    bt = content_matrix.T
    nsteps = pl.cdiv(k_dim, _BLK_K)
    tail = k_dim - (nsteps - 1) * _BLK_K
    body = functools.partial(_mm_kernel, nsteps=nsteps, tail=tail)
    out_t = pl.pallas_call(
        body,
        grid=(nsteps,),
        in_specs=[
            pl.BlockSpec((n, _BLK_K), lambda k: (0, k)),
            pl.BlockSpec((_BLK_K, m), lambda k: (k, 0)),
        ],
        out_specs=pl.BlockSpec((n, m), lambda k: (0, 0)),
        out_shape=jax.ShapeDtypeStruct((n, m), jnp.float32),
        compiler_params=pltpu.CompilerParams(
            dimension_semantics=("arbitrary",)
        ),
    )(bt, at)
    return out_t.T


# speedup vs baseline: 1.0156x; 1.0048x over previous
import functools

import jax
import jax.numpy as jnp
from jax import lax
from jax.experimental import pallas as pl
from jax.experimental.pallas import tpu as pltpu

_BLK_K = 2816


def _mm_kernel(bt_ref, at_ref, o_ref, *, nsteps, tail):
    k = pl.program_id(0)

    @pl.when(k == 0)
    def _init():
        o_ref[...] = jnp.zeros_like(o_ref)

    @pl.when(k < nsteps - 1)
    def _body():
        o_ref[...] += jnp.dot(
            bt_ref[...].astype(jnp.bfloat16),
            at_ref[...].astype(jnp.bfloat16),
            preferred_element_type=jnp.float32,
        )

    @pl.when(k == nsteps - 1)
    def _tail():
        bt = bt_ref[...]
        col = lax.broadcasted_iota(jnp.int32, bt.shape, 1)
        bt = jnp.where(col < tail, bt, 0.0)
        at = at_ref[...]
        row = lax.broadcasted_iota(jnp.int32, at.shape, 0)
        at = jnp.where(row < tail, at, 0.0)
        o_ref[...] += jnp.dot(
            bt.astype(jnp.bfloat16),
            at.astype(jnp.bfloat16),
            preferred_element_type=jnp.float32,
        )


def kernel(attention_weights, content_matrix):
    m, k_dim = attention_weights.shape
    _, n = content_matrix.shape
    at = attention_weights.T
    bt = content_matrix.T
    nsteps = pl.cdiv(k_dim, _BLK_K)
    tail = k_dim - (nsteps - 1) * _BLK_K
    body = functools.partial(_mm_kernel, nsteps=nsteps, tail=tail)
    out_t = pl.pallas_call(
        body,
        grid=(nsteps,),
        in_specs=[
            pl.BlockSpec((n, _BLK_K), lambda k: (0, k)),
            pl.BlockSpec((_BLK_K, m), lambda k: (k, 0)),
        ],
        out_specs=pl.BlockSpec((n, m), lambda k: (0, 0)),
        out_shape=jax.ShapeDtypeStruct((n, m), jnp.float32),
        compiler_params=pltpu.CompilerParams(
            dimension_semantics=("arbitrary",)
        ),
    )(bt, at)
    return out_t.T
